# baseline (speedup 1.0000x reference)
import functools

import jax
import jax.numpy as jnp
from jax import lax
from jax.experimental import pallas as pl
from jax.experimental.pallas import tpu as pltpu
from jax.experimental.pallas import tpu_sc as plsc

N = 10000
E = 320000
D = 128
DE = 16
NPAD = 10240
NC = 2
NS = 16
NW = NC * NS
CHUNK = 128
RT = 80
ER = NW * RT
EP = ER * CHUNK
RPT = NPAD // NS
NB = 2
ND = 8
EH = EP // 2
RH = RT // 2


def _sc_mesh():
    return plsc.VectorSubcoreMesh(core_axis_name="c", subcore_axis_name="s")


def _deg_body(dst2, ones_hbm, zeros_hbm, out_hbm, didx, ones_v,
              deg_sh, *ss):
    core = lax.axis_index("c")
    sub = lax.axis_index("s")
    tid = core * NS + sub
    row0 = tid * RT
    r0 = sub * RPT
    pltpu.sync_copy(zeros_hbm.at[pl.ds(r0, RPT)], deg_sh.at[pl.ds(r0, RPT)])
    pltpu.sync_copy(dst2.at[pl.ds(row0, RT)], didx)
    pltpu.sync_copy(ones_hbm, ones_v)
    plsc.subcore_barrier()

    def s_start(c, b):
        pltpu.async_copy(ones_v, deg_sh.at[didx.at[c]], ss[b], add=True)

    def s_wait(c, b):
        pltpu.make_async_copy(ones_v, deg_sh.at[didx.at[c]], ss[b]).wait()

    for b in range(ND):
        s_start(b, b)

    @pl.loop(0, RT // ND)
    def _(i):
        for b in range(ND):
            c = i * ND + b

            @pl.when(c + ND < RT)
            def _():
                s_wait(c, b)
                s_start(c + ND, b)

    for b in range(ND):
        s_wait(RT - ND + b, b)

    plsc.subcore_barrier()
    pltpu.sync_copy(deg_sh.at[pl.ds(r0, RPT)],
                    out_hbm.at[core, pl.ds(r0, RPT)])


def _deg(dst2, ones128, zeros128):
    return pl.kernel(
        _deg_body,
        out_type=jax.ShapeDtypeStruct((NC, NPAD, D), jnp.float32),
        mesh=_sc_mesh(),
        scratch_types=[
            pltpu.VMEM((RT, CHUNK), jnp.int32),
            pltpu.VMEM((CHUNK, D), jnp.float32),
            pltpu.VMEM_SHARED((NPAD, D), jnp.float32),
        ] + [pltpu.SemaphoreType.DMA] * ND,
    )(dst2, ones128, zeros128)


def _scat_body(h_hbm, src1, dst1, zeros_hbm, out_hbm,
               sidx, didx, rows, agg_sh, sem):
    core = lax.axis_index("c")
    sub = lax.axis_index("s")
    tid = core * NS + sub
    base = tid * RT * CHUNK
    r0 = sub * RPT
    pltpu.sync_copy(zeros_hbm.at[pl.ds(r0, RPT)], agg_sh.at[pl.ds(r0, RPT)])
    plsc.subcore_barrier()

    @pl.loop(0, RT)
    def _(c):
        off = base + c * CHUNK
        pltpu.sync_copy(src1.at[pl.ds(off, CHUNK)], sidx)
        pltpu.async_copy(h_hbm.at[sidx], rows, sem).wait()
        pltpu.sync_copy(dst1.at[pl.ds(off, CHUNK)], didx)
        pltpu.sync_copy(rows, agg_sh.at[didx], add=True)

    plsc.subcore_barrier()
    pltpu.sync_copy(agg_sh.at[pl.ds(r0, RPT)],
                    out_hbm.at[core, pl.ds(r0, RPT)])


def _scatter(h, src1, dst1, zeros128):
    return pl.kernel(
        _scat_body,
        out_type=jax.ShapeDtypeStruct((NC, NPAD, D), jnp.float32),
        mesh=_sc_mesh(),
        scratch_types=[
            pltpu.VMEM((CHUNK,), jnp.int32),
            pltpu.VMEM((CHUNK,), jnp.int32),
            pltpu.VMEM((CHUNK, D), jnp.float32),
            pltpu.VMEM_SHARED((NPAD, D), jnp.float32),
            pltpu.SemaphoreType.DMA,
        ],
    )(h, src1, dst1, zeros128)


def _v_add(r1, r2):
    @pl.loop(0, CHUNK)
    def _(r):
        for q in range(D // 16):
            sl = pl.ds(q * 16, 16)
            plsc.addupdate(r1.at[r, sl], r2[r, sl])


def _v_body(asrc, adst, src1, dst1, out_hbm,
            sidx, didx, rows1, rows2, sem1, sem2):
    core = lax.axis_index("c")
    sub = lax.axis_index("s")
    tid = core * NS + sub
    base = tid * RT * CHUNK

    @pl.loop(0, RT)
    def _(c):
        off = base + c * CHUNK
        pltpu.sync_copy(src1.at[pl.ds(off, CHUNK)], sidx)
        pltpu.sync_copy(dst1.at[pl.ds(off, CHUNK)], didx)
        d1 = pltpu.async_copy(asrc.at[sidx], rows1, sem1)
        d2 = pltpu.async_copy(adst.at[didx], rows2, sem2)
        d1.wait()
        d2.wait()
        _v_add(rows1, rows2)
        pltpu.sync_copy(rows1, out_hbm.at[pl.ds(off, CHUNK)])


def _vkern(a_src, a_dst, src1, dst1):
    return pl.kernel(
        _v_body,
        out_type=jax.ShapeDtypeStruct((EP, D), jnp.float32),
        mesh=_sc_mesh(),
        scratch_types=[
            pltpu.VMEM((CHUNK,), jnp.int32),
            pltpu.VMEM((CHUNK,), jnp.int32),
            pltpu.VMEM((CHUNK, D), jnp.float32),
            pltpu.VMEM((CHUNK, D), jnp.float32),
            pltpu.SemaphoreType.DMA,
            pltpu.SemaphoreType.DMA,
        ],
    )(a_src, a_dst, src1, dst1)


BM = 512
BE = 4000


def _enc_body(x_ref, w_ref, b_ref, o_ref):
    o_ref[...] = jnp.dot(x_ref[...], w_ref[...],
                         preferred_element_type=jnp.float32) + b_ref[...]


def _encode(x_pad, W, b_row):
    return pl.pallas_call(
        _enc_body,
        grid=(NPAD // BM,),
        in_specs=[
            pl.BlockSpec((BM, D), lambda i: (i, 0)),
            pl.BlockSpec((D, D), lambda i: (0, 0)),
            pl.BlockSpec((1, D), lambda i: (0, 0)),
        ],
        out_specs=pl.BlockSpec((BM, D), lambda i: (i, 0)),
        out_shape=jax.ShapeDtypeStruct((NPAD, D), jnp.float32),
    )(x_pad, W, b_row)


def _recip_body(d_ref, o_ref):
    deg = d_ref[0, :, 0:1] + d_ref[1, :, 0:1]
    o_ref[...] = 1.0 / jnp.maximum(deg, 1.0)


def _recip(deg2):
    return pl.pallas_call(
        _recip_body,
        grid=(NPAD // BM,),
        in_specs=[pl.BlockSpec((NC, BM, D), lambda i: (0, i, 0))],
        out_specs=pl.BlockSpec((BM, 1), lambda i: (i, 0)),
        out_shape=jax.ShapeDtypeStruct((NPAD, 1), jnp.float32),
    )(deg2)


def _layer_body(a_ref, r_ref, w_ref, b_ref, o_ref):
    z = (a_ref[0] + a_ref[1]) * r_ref[...]
    o_ref[...] = jax.nn.relu(
        jnp.dot(z, w_ref[...], preferred_element_type=jnp.float32)
        + b_ref[...])


def _layer(agg2, recip, W, b_row):
    return pl.pallas_call(
        _layer_body,
        grid=(NPAD // BM,),
        in_specs=[
            pl.BlockSpec((NC, BM, D), lambda i: (0, i, 0)),
            pl.BlockSpec((BM, 1), lambda i: (i, 0)),
            pl.BlockSpec((D, D), lambda i: (0, 0)),
            pl.BlockSpec((1, D), lambda i: (0, 0)),
        ],
        out_specs=pl.BlockSpec((BM, D), lambda i: (i, 0)),
        out_shape=jax.ShapeDtypeStruct((NPAD, D), jnp.float32),
    )(agg2, recip, W, b_row)


def _layer3_body(a_ref, r_ref, w_ref, b_ref, ws_ref, wd_ref, os_ref, od_ref):
    z = (a_ref[0] + a_ref[1]) * r_ref[...]
    h = jax.nn.relu(
        jnp.dot(z, w_ref[...], preferred_element_type=jnp.float32)
        + b_ref[...])
    os_ref[...] = jnp.dot(h, ws_ref[...], preferred_element_type=jnp.float32)
    od_ref[...] = jnp.dot(h, wd_ref[...], preferred_element_type=jnp.float32)


def _layer3(agg2, recip, W, b_row, W_src, W_dst):
    return pl.pallas_call(
        _layer3_body,
        grid=(NPAD // BM,),
        in_specs=[
            pl.BlockSpec((NC, BM, D), lambda i: (0, i, 0)),
            pl.BlockSpec((BM, 1), lambda i: (i, 0)),
            pl.BlockSpec((D, D), lambda i: (0, 0)),
            pl.BlockSpec((1, D), lambda i: (0, 0)),
            pl.BlockSpec((D, D), lambda i: (0, 0)),
            pl.BlockSpec((D, D), lambda i: (0, 0)),
        ],
        out_specs=[
            pl.BlockSpec((BM, D), lambda i: (i, 0)),
            pl.BlockSpec((BM, D), lambda i: (i, 0)),
        ],
        out_shape=[
            jax.ShapeDtypeStruct((NPAD, D), jnp.float32),
            jax.ShapeDtypeStruct((NPAD, D), jnp.float32),
        ],
    )(agg2, recip, W, b_row, W_src, W_dst)


def _score_body(v_ref, e_ref, we_ref, be_ref, wo_ref, bo_ref, o_ref):
    eh = jnp.dot(e_ref[...], we_ref[...],
                 preferred_element_type=jnp.float32) + be_ref[...]
    t = jax.nn.relu(v_ref[...] + eh)
    o_ref[...] = jnp.sum(t * wo_ref[...], axis=1, keepdims=True) + bo_ref[...]


def _score(V, e_full, W_edge, be_row, wo_row, bo_11):
    return pl.pallas_call(
        _score_body,
        grid=(E // BE,),
        in_specs=[
            pl.BlockSpec((BE, D), lambda i: (i, 0)),
            pl.BlockSpec((BE, DE), lambda i: (i, 0)),
            pl.BlockSpec((DE, D), lambda i: (0, 0)),
            pl.BlockSpec((1, D), lambda i: (0, 0)),
            pl.BlockSpec((1, D), lambda i: (0, 0)),
            pl.BlockSpec((1, 1), lambda i: (0, 0)),
        ],
        out_specs=pl.BlockSpec((BE, 1), lambda i: (i, 0)),
        out_shape=jax.ShapeDtypeStruct((E, 1), jnp.float32),
    )(V, e_full, W_edge, be_row, wo_row, bo_11)


def kernel(x, edge_index, e, W_node, b_node, W_edge, b_edge,
           W_gcn0, b_gcn0, W_gcn1, b_gcn1, W_gcn2, b_gcn2,
           W_src, W_dst, w_out, b_out):
    pad = jnp.full((EP - E,), N, jnp.int32)
    src1 = jnp.concatenate([edge_index[0].astype(jnp.int32), pad])
    dst1 = jnp.concatenate([edge_index[1].astype(jnp.int32), pad])
    src2 = src1.reshape(ER, CHUNK)
    dst2 = dst1.reshape(ER, CHUNK)
    x_pad = jnp.pad(x, ((0, NPAD - N), (0, 0)))
    zeros128 = jnp.zeros((NPAD, D), jnp.float32)
    ones128 = jnp.ones((CHUNK, D), jnp.float32)

    deg2 = _deg(dst2, ones128, zeros128)
    h = _encode(x_pad, W_node, b_node.reshape(1, D))
    recip = _recip(deg2)
    for W, b in ((W_gcn0, b_gcn0), (W_gcn1, b_gcn1)):
        agg2 = _scatter(h, src1, dst1, zeros128)
        h = _layer(agg2, recip, W, b.reshape(1, D))
    agg2 = _scatter(h, src1, dst1, zeros128)
    a_src, a_dst = _layer3(agg2, recip, W_gcn2, b_gcn2.reshape(1, D),
                           W_src, W_dst)

    V = _vkern(a_src, a_dst, src1, dst1)
    scores = _score(V, e, W_edge, b_edge.reshape(1, D),
                    w_out.reshape(1, D), b_out.reshape(1, 1))
    return scores[:, 0]


# speedup vs baseline: 1.9574x; 1.9574x over previous
import functools

import jax
import jax.numpy as jnp
from jax import lax
from jax.experimental import pallas as pl
from jax.experimental.pallas import tpu as pltpu
from jax.experimental.pallas import tpu_sc as plsc

N = 10000
E = 320000
D = 128
DE = 16
NPAD = 10240
NC = 2
NS = 16
NW = NC * NS
CHUNK = 128
RT = 80
ER = NW * RT
EP = ER * CHUNK
RPT = NPAD // NS
NB = 2
ND = 8
EH = EP // 2
RH = RT // 2


def _sc_mesh():
    return plsc.VectorSubcoreMesh(core_axis_name="c", subcore_axis_name="s")


def _deg_body(dst2, ones_hbm, zeros_hbm, out_hbm, didx, ones_v,
              deg_sh, *ss):
    core = lax.axis_index("c")
    sub = lax.axis_index("s")
    tid = core * NS + sub
    row0 = tid * RT
    r0 = sub * RPT
    pltpu.sync_copy(zeros_hbm.at[pl.ds(r0, RPT)], deg_sh.at[pl.ds(r0, RPT)])
    pltpu.sync_copy(dst2.at[pl.ds(row0, RT)], didx)
    pltpu.sync_copy(ones_hbm, ones_v)
    plsc.subcore_barrier()

    def s_start(c, b):
        pltpu.async_copy(ones_v, deg_sh.at[didx.at[c]], ss[b], add=True)

    def s_wait(c, b):
        pltpu.make_async_copy(ones_v, deg_sh.at[didx.at[c]], ss[b]).wait()

    for b in range(ND):
        s_start(b, b)

    @pl.loop(0, RT // ND)
    def _(i):
        for b in range(ND):
            c = i * ND + b

            @pl.when(c + ND < RT)
            def _():
                s_wait(c, b)
                s_start(c + ND, b)

    for b in range(ND):
        s_wait(RT - ND + b, b)

    plsc.subcore_barrier()
    pltpu.sync_copy(deg_sh.at[pl.ds(r0, RPT)],
                    out_hbm.at[core, pl.ds(r0, RPT)])


def _deg(dst2, ones128, zeros128):
    return pl.kernel(
        _deg_body,
        out_type=jax.ShapeDtypeStruct((NC, NPAD, D), jnp.float32),
        mesh=_sc_mesh(),
        scratch_types=[
            pltpu.VMEM((RT, CHUNK), jnp.int32),
            pltpu.VMEM((CHUNK, D), jnp.float32),
            pltpu.VMEM_SHARED((NPAD, D), jnp.float32),
        ] + [pltpu.SemaphoreType.DMA] * ND,
    )(dst2, ones128, zeros128)


def _scat_body(h_hbm, src1, dst1, zeros_hbm, out_hbm,
               sidx, didx, rows, agg_sh, sem):
    core = lax.axis_index("c")
    sub = lax.axis_index("s")
    tid = core * NS + sub
    base = tid * RT * CHUNK
    r0 = sub * RPT
    pltpu.sync_copy(zeros_hbm.at[pl.ds(r0, RPT)], agg_sh.at[pl.ds(r0, RPT)])
    plsc.subcore_barrier()

    @pl.loop(0, RT)
    def _(c):
        off = base + c * CHUNK
        pltpu.sync_copy(src1.at[pl.ds(off, CHUNK)], sidx)
        pltpu.async_copy(h_hbm.at[sidx], rows, sem).wait()
        pltpu.sync_copy(dst1.at[pl.ds(off, CHUNK)], didx)
        pltpu.sync_copy(rows, agg_sh.at[didx], add=True)

    plsc.subcore_barrier()
    pltpu.sync_copy(agg_sh.at[pl.ds(r0, RPT)],
                    out_hbm.at[core, pl.ds(r0, RPT)])


def _scatter(h, src1, dst1, zeros128):
    return pl.kernel(
        _scat_body,
        out_type=jax.ShapeDtypeStruct((NC, NPAD, D), jnp.float32),
        mesh=_sc_mesh(),
        scratch_types=[
            pltpu.VMEM((CHUNK,), jnp.int32),
            pltpu.VMEM((CHUNK,), jnp.int32),
            pltpu.VMEM((CHUNK, D), jnp.float32),
            pltpu.VMEM_SHARED((NPAD, D), jnp.float32),
            pltpu.SemaphoreType.DMA,
        ],
    )(h, src1, dst1, zeros128)


def _v_add(r1, r2):
    @pl.loop(0, CHUNK)
    def _(r):
        for q in range(D // 16):
            sl = pl.ds(q * 16, 16)
            plsc.addupdate(r1.at[r, sl], r2[r, sl])


def _v_body(asrc, adst, src1, dst1, out_hbm,
            sidx, didx, rows1, rows2, sem1, sem2):
    core = lax.axis_index("c")
    sub = lax.axis_index("s")
    tid = core * NS + sub
    base = tid * RT * CHUNK

    @pl.loop(0, RT)
    def _(c):
        off = base + c * CHUNK
        pltpu.sync_copy(src1.at[pl.ds(off, CHUNK)], sidx)
        pltpu.sync_copy(dst1.at[pl.ds(off, CHUNK)], didx)
        d1 = pltpu.async_copy(asrc.at[sidx], rows1, sem1)
        d2 = pltpu.async_copy(adst.at[didx], rows2, sem2)
        d1.wait()
        d2.wait()
        _v_add(rows1, rows2)
        pltpu.sync_copy(rows1, out_hbm.at[pl.ds(off, CHUNK)])


def _vkern(a_src, a_dst, src1, dst1):
    return pl.kernel(
        _v_body,
        out_type=jax.ShapeDtypeStruct((EP, D), jnp.float32),
        mesh=_sc_mesh(),
        scratch_types=[
            pltpu.VMEM((CHUNK,), jnp.int32),
            pltpu.VMEM((CHUNK,), jnp.int32),
            pltpu.VMEM((CHUNK, D), jnp.float32),
            pltpu.VMEM((CHUNK, D), jnp.float32),
            pltpu.SemaphoreType.DMA,
            pltpu.SemaphoreType.DMA,
        ],
    )(a_src, a_dst, src1, dst1)


BM = 512
BE = 4000


def _enc_body(x_ref, w_ref, b_ref, o_ref):
    o_ref[...] = jnp.dot(x_ref[...], w_ref[...],
                         preferred_element_type=jnp.float32) + b_ref[...]


def _encode(x_pad, W, b_row):
    return pl.pallas_call(
        _enc_body,
        grid=(NPAD // BM,),
        in_specs=[
            pl.BlockSpec((BM, D), lambda i: (i, 0)),
            pl.BlockSpec((D, D), lambda i: (0, 0)),
            pl.BlockSpec((1, D), lambda i: (0, 0)),
        ],
        out_specs=pl.BlockSpec((BM, D), lambda i: (i, 0)),
        out_shape=jax.ShapeDtypeStruct((NPAD, D), jnp.float32),
    )(x_pad, W, b_row)


def _recip_body(d_ref, o_ref):
    deg = d_ref[0, :, 0:1] + d_ref[1, :, 0:1]
    o_ref[...] = 1.0 / jnp.maximum(deg, 1.0)


def _recip(deg2):
    return pl.pallas_call(
        _recip_body,
        grid=(NPAD // BM,),
        in_specs=[pl.BlockSpec((NC, BM, D), lambda i: (0, i, 0))],
        out_specs=pl.BlockSpec((BM, 1), lambda i: (i, 0)),
        out_shape=jax.ShapeDtypeStruct((NPAD, 1), jnp.float32),
    )(deg2)


def _layer_body(a_ref, r_ref, w_ref, b_ref, o_ref):
    z = (a_ref[0] + a_ref[1]) * r_ref[...]
    o_ref[...] = jax.nn.relu(
        jnp.dot(z, w_ref[...], preferred_element_type=jnp.float32)
        + b_ref[...])


def _layer(agg2, recip, W, b_row):
    return pl.pallas_call(
        _layer_body,
        grid=(NPAD // BM,),
        in_specs=[
            pl.BlockSpec((NC, BM, D), lambda i: (0, i, 0)),
            pl.BlockSpec((BM, 1), lambda i: (i, 0)),
            pl.BlockSpec((D, D), lambda i: (0, 0)),
            pl.BlockSpec((1, D), lambda i: (0, 0)),
        ],
        out_specs=pl.BlockSpec((BM, D), lambda i: (i, 0)),
        out_shape=jax.ShapeDtypeStruct((NPAD, D), jnp.float32),
    )(agg2, recip, W, b_row)


def _layer3_body(a_ref, r_ref, w_ref, b_ref, ws_ref, wd_ref, os_ref, od_ref):
    z = (a_ref[0] + a_ref[1]) * r_ref[...]
    h = jax.nn.relu(
        jnp.dot(z, w_ref[...], preferred_element_type=jnp.float32)
        + b_ref[...])
    os_ref[...] = jnp.dot(h, ws_ref[...], preferred_element_type=jnp.float32)
    od_ref[...] = jnp.dot(h, wd_ref[...], preferred_element_type=jnp.float32)


def _layer3(agg2, recip, W, b_row, W_src, W_dst):
    return pl.pallas_call(
        _layer3_body,
        grid=(NPAD // BM,),
        in_specs=[
            pl.BlockSpec((NC, BM, D), lambda i: (0, i, 0)),
            pl.BlockSpec((BM, 1), lambda i: (i, 0)),
            pl.BlockSpec((D, D), lambda i: (0, 0)),
            pl.BlockSpec((1, D), lambda i: (0, 0)),
            pl.BlockSpec((D, D), lambda i: (0, 0)),
            pl.BlockSpec((D, D), lambda i: (0, 0)),
        ],
        out_specs=[
            pl.BlockSpec((BM, D), lambda i: (i, 0)),
            pl.BlockSpec((BM, D), lambda i: (i, 0)),
        ],
        out_shape=[
            jax.ShapeDtypeStruct((NPAD, D), jnp.float32),
            jax.ShapeDtypeStruct((NPAD, D), jnp.float32),
        ],
    )(agg2, recip, W, b_row, W_src, W_dst)


def _score_body(v_ref, e_ref, we_ref, be_ref, wo_ref, bo_ref, o_ref):
    eh = jnp.dot(e_ref[...], we_ref[...],
                 preferred_element_type=jnp.float32) + be_ref[...]
    t = jax.nn.relu(v_ref[...] + eh)
    o_ref[...] = jnp.sum(t * wo_ref[...], axis=1, keepdims=True) + bo_ref[...]


def _score(V, e_full, W_edge, be_row, wo_row, bo_11):
    return pl.pallas_call(
        _score_body,
        grid=(E // BE,),
        in_specs=[
            pl.BlockSpec((BE, D), lambda i: (i, 0)),
            pl.BlockSpec((BE, DE), lambda i: (i, 0)),
            pl.BlockSpec((DE, D), lambda i: (0, 0)),
            pl.BlockSpec((1, D), lambda i: (0, 0)),
            pl.BlockSpec((1, D), lambda i: (0, 0)),
            pl.BlockSpec((1, 1), lambda i: (0, 0)),
        ],
        out_specs=pl.BlockSpec((BE, 1), lambda i: (i, 0)),
        out_shape=jax.ShapeDtypeStruct((E, 1), jnp.float32),
    )(V, e_full, W_edge, be_row, wo_row, bo_11)


def kernel(x, edge_index, e, W_node, b_node, W_edge, b_edge,
           W_gcn0, b_gcn0, W_gcn1, b_gcn1, W_gcn2, b_gcn2,
           W_src, W_dst, w_out, b_out):
    pad = N + jnp.arange(EP - E, dtype=jnp.int32) % (NPAD - N)
    src1 = jnp.concatenate([edge_index[0].astype(jnp.int32), pad])
    dst1 = jnp.concatenate([edge_index[1].astype(jnp.int32), pad])
    src2 = src1.reshape(ER, CHUNK)
    dst2 = dst1.reshape(ER, CHUNK)
    x_pad = jnp.pad(x, ((0, NPAD - N), (0, 0)))
    zeros128 = jnp.zeros((NPAD, D), jnp.float32)
    ones128 = jnp.ones((CHUNK, D), jnp.float32)

    deg2 = _deg(dst2, ones128, zeros128)
    h = _encode(x_pad, W_node, b_node.reshape(1, D))
    recip = _recip(deg2)
    for W, b in ((W_gcn0, b_gcn0), (W_gcn1, b_gcn1)):
        agg2 = _scatter(h, src1, dst1, zeros128)
        h = _layer(agg2, recip, W, b.reshape(1, D))
    agg2 = _scatter(h, src1, dst1, zeros128)
    a_src, a_dst = _layer3(agg2, recip, W_gcn2, b_gcn2.reshape(1, D),
                           W_src, W_dst)

    V = _vkern(a_src, a_dst, src1, dst1)
    scores = _score(V, e, W_edge, b_edge.reshape(1, D),
                    w_out.reshape(1, D), b_out.reshape(1, 1))
    return scores[:, 0]


# speedup vs baseline: 2.7135x; 1.3863x over previous
import functools

import jax
import jax.numpy as jnp
from jax import lax
from jax.experimental import pallas as pl
from jax.experimental.pallas import tpu as pltpu
from jax.experimental.pallas import tpu_sc as plsc

N = 10000
E = 320000
D = 128
DE = 16
NPAD = 10240
NC = 2
NS = 16
NW = NC * NS
CHUNK = 128
RT = 80
ER = NW * RT
EP = ER * CHUNK
RPT = NPAD // NS
NB = 2
ND = 8
EH = EP // 2
RH = RT // 2


def _sc_mesh():
    return plsc.VectorSubcoreMesh(core_axis_name="c", subcore_axis_name="s")


def _deg_body(dst2, ones_hbm, zeros_hbm, out_hbm, didx, ones_v,
              deg_sh, *ss):
    core = lax.axis_index("c")
    sub = lax.axis_index("s")
    tid = core * NS + sub
    row0 = tid * RT
    r0 = sub * RPT
    pltpu.sync_copy(zeros_hbm.at[pl.ds(r0, RPT)], deg_sh.at[pl.ds(r0, RPT)])
    pltpu.sync_copy(dst2.at[pl.ds(row0, RT)], didx)
    pltpu.sync_copy(ones_hbm, ones_v)
    plsc.subcore_barrier()

    def s_start(c, b):
        pltpu.async_copy(ones_v, deg_sh.at[didx.at[c]], ss[b], add=True)

    def s_wait(c, b):
        pltpu.make_async_copy(ones_v, deg_sh.at[didx.at[c]], ss[b]).wait()

    for b in range(ND):
        s_start(b, b)

    @pl.loop(0, RT // ND)
    def _(i):
        for b in range(ND):
            c = i * ND + b

            @pl.when(c + ND < RT)
            def _():
                s_wait(c, b)
                s_start(c + ND, b)

    for b in range(ND):
        s_wait(RT - ND + b, b)

    plsc.subcore_barrier()
    pltpu.sync_copy(deg_sh.at[pl.ds(r0, RPT)],
                    out_hbm.at[core, pl.ds(r0, RPT)])


def _deg(dst2, ones128, zeros128):
    return pl.kernel(
        _deg_body,
        out_type=jax.ShapeDtypeStruct((NC, NPAD, D), jnp.float32),
        mesh=_sc_mesh(),
        scratch_types=[
            pltpu.VMEM((RT, CHUNK), jnp.int32),
            pltpu.VMEM((CHUNK, D), jnp.float32),
            pltpu.VMEM_SHARED((NPAD, D), jnp.float32),
        ] + [pltpu.SemaphoreType.DMA] * ND,
    )(dst2, ones128, zeros128)


def _scat_body(h_hbm, src2, dst1, zeros_hbm, out_hbm,
               sidx, didx0, didx1, rows, agg_sh, *sems):
    didx = (didx0, didx1)
    ii = sems[:NB]
    gs = sems[NB:2 * NB]
    ss = sems[2 * NB:]
    core = lax.axis_index("c")
    sub = lax.axis_index("s")
    tid = core * NS + sub
    row0 = tid * RT
    base = tid * RT * CHUNK
    r0 = sub * RPT
    pltpu.sync_copy(zeros_hbm.at[pl.ds(r0, RPT)], agg_sh.at[pl.ds(r0, RPT)])
    pltpu.sync_copy(src2.at[pl.ds(row0, RT)], sidx)
    plsc.subcore_barrier()

    def di_start(c, b):
        pltpu.async_copy(dst1.at[pl.ds(base + c * CHUNK, CHUNK)], didx[b],
                         ii[b])

    def di_wait(b):
        pltpu.make_async_copy(dst1.at[pl.ds(0, CHUNK)], didx[b], ii[b]).wait()

    def g_start(c, b):
        pltpu.async_copy(h_hbm.at[sidx.at[c]], rows.at[b], gs[b])

    def g_wait(b):
        pltpu.make_async_copy(h_hbm.at[pl.ds(0, CHUNK)], rows.at[b],
                              gs[b]).wait()

    def s_start(b):
        pltpu.async_copy(rows.at[b], agg_sh.at[didx[b]], ss[b], add=True)

    def s_wait(b):
        pltpu.make_async_copy(rows.at[b], agg_sh.at[pl.ds(0, CHUNK)],
                              ss[b]).wait()

    for b in range(NB):
        di_start(b, b)
        g_start(b, b)

    @pl.loop(0, RT // NB)
    def _(i):
        for b in range(NB):
            c = i * NB + b
            g_wait(b)
            di_wait(b)
            s_start(b)

            @pl.when(c + NB < RT)
            def _():
                s_wait(b)
                g_start(c + NB, b)
                di_start(c + NB, b)

    for b in range(NB):
        s_wait(b)

    plsc.subcore_barrier()
    pltpu.sync_copy(agg_sh.at[pl.ds(r0, RPT)],
                    out_hbm.at[core, pl.ds(r0, RPT)])


def _scatter(h, src2, dst1, zeros128):
    return pl.kernel(
        _scat_body,
        out_type=jax.ShapeDtypeStruct((NC, NPAD, D), jnp.float32),
        mesh=_sc_mesh(),
        scratch_types=[
            pltpu.VMEM((RT, CHUNK), jnp.int32),
            pltpu.VMEM((CHUNK,), jnp.int32),
            pltpu.VMEM((CHUNK,), jnp.int32),
            pltpu.VMEM((NB, CHUNK, D), jnp.float32),
            pltpu.VMEM_SHARED((NPAD, D), jnp.float32),
        ] + [pltpu.SemaphoreType.DMA] * (3 * NB),
    )(h, src2, dst1, zeros128)


def _v_add(r1, r2):
    @pl.loop(0, CHUNK)
    def _(r):
        for q in range(D // 16):
            sl = pl.ds(q * 16, 16)
            plsc.addupdate(r1.at[r, sl], r2[r, sl])


def _v_body(asrc, adst, src1, dst1, out_hbm,
            sidx, didx, rows1, rows2, sem1, sem2):
    core = lax.axis_index("c")
    sub = lax.axis_index("s")
    tid = core * NS + sub
    base = tid * RT * CHUNK

    @pl.loop(0, RT)
    def _(c):
        off = base + c * CHUNK
        pltpu.sync_copy(src1.at[pl.ds(off, CHUNK)], sidx)
        pltpu.sync_copy(dst1.at[pl.ds(off, CHUNK)], didx)
        d1 = pltpu.async_copy(asrc.at[sidx], rows1, sem1)
        d2 = pltpu.async_copy(adst.at[didx], rows2, sem2)
        d1.wait()
        d2.wait()
        _v_add(rows1, rows2)
        pltpu.sync_copy(rows1, out_hbm.at[pl.ds(off, CHUNK)])


def _vkern(a_src, a_dst, src1, dst1):
    return pl.kernel(
        _v_body,
        out_type=jax.ShapeDtypeStruct((EP, D), jnp.float32),
        mesh=_sc_mesh(),
        scratch_types=[
            pltpu.VMEM((CHUNK,), jnp.int32),
            pltpu.VMEM((CHUNK,), jnp.int32),
            pltpu.VMEM((CHUNK, D), jnp.float32),
            pltpu.VMEM((CHUNK, D), jnp.float32),
            pltpu.SemaphoreType.DMA,
            pltpu.SemaphoreType.DMA,
        ],
    )(a_src, a_dst, src1, dst1)


BM = 512
BE = 4000


def _enc_body(x_ref, w_ref, b_ref, o_ref):
    o_ref[...] = jnp.dot(x_ref[...], w_ref[...],
                         preferred_element_type=jnp.float32) + b_ref[...]


def _encode(x_pad, W, b_row):
    return pl.pallas_call(
        _enc_body,
        grid=(NPAD // BM,),
        in_specs=[
            pl.BlockSpec((BM, D), lambda i: (i, 0)),
            pl.BlockSpec((D, D), lambda i: (0, 0)),
            pl.BlockSpec((1, D), lambda i: (0, 0)),
        ],
        out_specs=pl.BlockSpec((BM, D), lambda i: (i, 0)),
        out_shape=jax.ShapeDtypeStruct((NPAD, D), jnp.float32),
    )(x_pad, W, b_row)


def _recip_body(d_ref, o_ref):
    deg = d_ref[0, :, 0:1] + d_ref[1, :, 0:1]
    o_ref[...] = 1.0 / jnp.maximum(deg, 1.0)


def _recip(deg2):
    return pl.pallas_call(
        _recip_body,
        grid=(NPAD // BM,),
        in_specs=[pl.BlockSpec((NC, BM, D), lambda i: (0, i, 0))],
        out_specs=pl.BlockSpec((BM, 1), lambda i: (i, 0)),
        out_shape=jax.ShapeDtypeStruct((NPAD, 1), jnp.float32),
    )(deg2)


def _layer_body(a_ref, r_ref, w_ref, b_ref, o_ref):
    z = (a_ref[0] + a_ref[1]) * r_ref[...]
    o_ref[...] = jax.nn.relu(
        jnp.dot(z, w_ref[...], preferred_element_type=jnp.float32)
        + b_ref[...])


def _layer(agg2, recip, W, b_row):
    return pl.pallas_call(
        _layer_body,
        grid=(NPAD // BM,),
        in_specs=[
            pl.BlockSpec((NC, BM, D), lambda i: (0, i, 0)),
            pl.BlockSpec((BM, 1), lambda i: (i, 0)),
            pl.BlockSpec((D, D), lambda i: (0, 0)),
            pl.BlockSpec((1, D), lambda i: (0, 0)),
        ],
        out_specs=pl.BlockSpec((BM, D), lambda i: (i, 0)),
        out_shape=jax.ShapeDtypeStruct((NPAD, D), jnp.float32),
    )(agg2, recip, W, b_row)


def _layer3_body(a_ref, r_ref, w_ref, b_ref, ws_ref, wd_ref, os_ref, od_ref):
    z = (a_ref[0] + a_ref[1]) * r_ref[...]
    h = jax.nn.relu(
        jnp.dot(z, w_ref[...], preferred_element_type=jnp.float32)
        + b_ref[...])
    os_ref[...] = jnp.dot(h, ws_ref[...], preferred_element_type=jnp.float32)
    od_ref[...] = jnp.dot(h, wd_ref[...], preferred_element_type=jnp.float32)


def _layer3(agg2, recip, W, b_row, W_src, W_dst):
    return pl.pallas_call(
        _layer3_body,
        grid=(NPAD // BM,),
        in_specs=[
            pl.BlockSpec((NC, BM, D), lambda i: (0, i, 0)),
            pl.BlockSpec((BM, 1), lambda i: (i, 0)),
            pl.BlockSpec((D, D), lambda i: (0, 0)),
            pl.BlockSpec((1, D), lambda i: (0, 0)),
            pl.BlockSpec((D, D), lambda i: (0, 0)),
            pl.BlockSpec((D, D), lambda i: (0, 0)),
        ],
        out_specs=[
            pl.BlockSpec((BM, D), lambda i: (i, 0)),
            pl.BlockSpec((BM, D), lambda i: (i, 0)),
        ],
        out_shape=[
            jax.ShapeDtypeStruct((NPAD, D), jnp.float32),
            jax.ShapeDtypeStruct((NPAD, D), jnp.float32),
        ],
    )(agg2, recip, W, b_row, W_src, W_dst)


def _score_body(v_ref, e_ref, we_ref, be_ref, wo_ref, bo_ref, o_ref):
    eh = jnp.dot(e_ref[...], we_ref[...],
                 preferred_element_type=jnp.float32) + be_ref[...]
    t = jax.nn.relu(v_ref[...] + eh)
    o_ref[...] = jnp.sum(t * wo_ref[...], axis=1, keepdims=True) + bo_ref[...]


def _score(V, e_full, W_edge, be_row, wo_row, bo_11):
    return pl.pallas_call(
        _score_body,
        grid=(E // BE,),
        in_specs=[
            pl.BlockSpec((BE, D), lambda i: (i, 0)),
            pl.BlockSpec((BE, DE), lambda i: (i, 0)),
            pl.BlockSpec((DE, D), lambda i: (0, 0)),
            pl.BlockSpec((1, D), lambda i: (0, 0)),
            pl.BlockSpec((1, D), lambda i: (0, 0)),
            pl.BlockSpec((1, 1), lambda i: (0, 0)),
        ],
        out_specs=pl.BlockSpec((BE, 1), lambda i: (i, 0)),
        out_shape=jax.ShapeDtypeStruct((E, 1), jnp.float32),
    )(V, e_full, W_edge, be_row, wo_row, bo_11)


def kernel(x, edge_index, e, W_node, b_node, W_edge, b_edge,
           W_gcn0, b_gcn0, W_gcn1, b_gcn1, W_gcn2, b_gcn2,
           W_src, W_dst, w_out, b_out):
    pad = N + jnp.arange(EP - E, dtype=jnp.int32) % (NPAD - N)
    src1 = jnp.concatenate([edge_index[0].astype(jnp.int32), pad])
    dst1 = jnp.concatenate([edge_index[1].astype(jnp.int32), pad])
    src2 = src1.reshape(ER, CHUNK)
    dst2 = dst1.reshape(ER, CHUNK)
    x_pad = jnp.pad(x, ((0, NPAD - N), (0, 0)))
    zeros128 = jnp.zeros((NPAD, D), jnp.float32)
    ones128 = jnp.ones((CHUNK, D), jnp.float32)

    deg2 = _deg(dst2, ones128, zeros128)
    h = _encode(x_pad, W_node, b_node.reshape(1, D))
    recip = _recip(deg2)
    for W, b in ((W_gcn0, b_gcn0), (W_gcn1, b_gcn1)):
        agg2 = _scatter(h, src2, dst1, zeros128)
        h = _layer(agg2, recip, W, b.reshape(1, D))
    agg2 = _scatter(h, src2, dst1, zeros128)
    a_src, a_dst = _layer3(agg2, recip, W_gcn2, b_gcn2.reshape(1, D),
                           W_src, W_dst)

    V = _vkern(a_src, a_dst, src1, dst1)
    scores = _score(V, e, W_edge, b_edge.reshape(1, D),
                    w_out.reshape(1, D), b_out.reshape(1, 1))
    return scores[:, 0]


# speedup vs baseline: 3.1796x; 1.1718x over previous
import functools

import jax
import jax.numpy as jnp
from jax import lax
from jax.experimental import pallas as pl
from jax.experimental.pallas import tpu as pltpu
from jax.experimental.pallas import tpu_sc as plsc

N = 10000
E = 320000
D = 128
DE = 16
NPAD = 10240
NC = 2
NS = 16
NW = NC * NS
CHUNK = 128
RT = 80
ER = NW * RT
EP = ER * CHUNK
RPT = NPAD // NS
NB = 2
ND = 8
NV = 2
EH = EP // 2
RH = RT // 2


def _sc_mesh():
    return plsc.VectorSubcoreMesh(core_axis_name="c", subcore_axis_name="s")


def _deg_body(dst2, ones_hbm, zeros_hbm, out_hbm, didx, ones_v,
              deg_sh, *ss):
    core = lax.axis_index("c")
    sub = lax.axis_index("s")
    tid = core * NS + sub
    row0 = tid * RT
    r0 = sub * RPT
    pltpu.sync_copy(zeros_hbm.at[pl.ds(r0, RPT)], deg_sh.at[pl.ds(r0, RPT)])
    pltpu.sync_copy(dst2.at[pl.ds(row0, RT)], didx)
    pltpu.sync_copy(ones_hbm, ones_v)
    plsc.subcore_barrier()

    def s_start(c, b):
        pltpu.async_copy(ones_v, deg_sh.at[didx.at[c]], ss[b], add=True)

    def s_wait(c, b):
        pltpu.make_async_copy(ones_v, deg_sh.at[didx.at[c]], ss[b]).wait()

    for b in range(ND):
        s_start(b, b)

    @pl.loop(0, RT // ND)
    def _(i):
        for b in range(ND):
            c = i * ND + b

            @pl.when(c + ND < RT)
            def _():
                s_wait(c, b)
                s_start(c + ND, b)

    for b in range(ND):
        s_wait(RT - ND + b, b)

    plsc.subcore_barrier()
    pltpu.sync_copy(deg_sh.at[pl.ds(r0, RPT)],
                    out_hbm.at[core, pl.ds(r0, RPT)])


def _deg(dst2, ones128, zeros128):
    return pl.kernel(
        _deg_body,
        out_type=jax.ShapeDtypeStruct((NC, NPAD, D), jnp.float32),
        mesh=_sc_mesh(),
        scratch_types=[
            pltpu.VMEM((RT, CHUNK), jnp.int32),
            pltpu.VMEM((CHUNK, D), jnp.float32),
            pltpu.VMEM_SHARED((NPAD, D), jnp.float32),
        ] + [pltpu.SemaphoreType.DMA] * ND,
    )(dst2, ones128, zeros128)


def _scat_body(h_hbm, src2, dst1, zeros_hbm, out_hbm,
               sidx, didx0, didx1, rows, agg_sh, *sems):
    didx = (didx0, didx1)
    ii = sems[:NB]
    gs = sems[NB:2 * NB]
    ss = sems[2 * NB:]
    core = lax.axis_index("c")
    sub = lax.axis_index("s")
    tid = core * NS + sub
    row0 = tid * RT
    base = tid * RT * CHUNK
    r0 = sub * RPT
    pltpu.sync_copy(zeros_hbm.at[pl.ds(r0, RPT)], agg_sh.at[pl.ds(r0, RPT)])
    pltpu.sync_copy(src2.at[pl.ds(row0, RT)], sidx)
    plsc.subcore_barrier()

    def di_start(c, b):
        pltpu.async_copy(dst1.at[pl.ds(base + c * CHUNK, CHUNK)], didx[b],
                         ii[b])

    def di_wait(b):
        pltpu.make_async_copy(dst1.at[pl.ds(0, CHUNK)], didx[b], ii[b]).wait()

    def g_start(c, b):
        pltpu.async_copy(h_hbm.at[sidx.at[c]], rows.at[b], gs[b])

    def g_wait(b):
        pltpu.make_async_copy(h_hbm.at[pl.ds(0, CHUNK)], rows.at[b],
                              gs[b]).wait()

    def s_start(b):
        pltpu.async_copy(rows.at[b], agg_sh.at[didx[b]], ss[b], add=True)

    def s_wait(b):
        pltpu.make_async_copy(rows.at[b], agg_sh.at[pl.ds(0, CHUNK)],
                              ss[b]).wait()

    for b in range(NB):
        di_start(b, b)
        g_start(b, b)

    @pl.loop(0, RT // NB)
    def _(i):
        for b in range(NB):
            c = i * NB + b
            g_wait(b)
            di_wait(b)
            s_start(b)

            @pl.when(c + NB < RT)
            def _():
                s_wait(b)
                g_start(c + NB, b)
                di_start(c + NB, b)

    for b in range(NB):
        s_wait(b)

    plsc.subcore_barrier()
    pltpu.sync_copy(agg_sh.at[pl.ds(r0, RPT)],
                    out_hbm.at[core, pl.ds(r0, RPT)])


def _scatter(h, src2, dst1, zeros128):
    return pl.kernel(
        _scat_body,
        out_type=jax.ShapeDtypeStruct((NC, NPAD, D), jnp.float32),
        mesh=_sc_mesh(),
        scratch_types=[
            pltpu.VMEM((RT, CHUNK), jnp.int32),
            pltpu.VMEM((CHUNK,), jnp.int32),
            pltpu.VMEM((CHUNK,), jnp.int32),
            pltpu.VMEM((NB, CHUNK, D), jnp.float32),
            pltpu.VMEM_SHARED((NPAD, D), jnp.float32),
        ] + [pltpu.SemaphoreType.DMA] * (3 * NB),
    )(h, src2, dst1, zeros128)


def _v_add(r1, r2):
    @pl.loop(0, CHUNK)
    def _(r):
        for q in range(D // 16):
            sl = pl.ds(q * 16, 16)
            plsc.addupdate(r1.at[r, sl], r2[r, sl])


def _v_body(asrc, adst, src2, dst2, out_hbm,
            sidx, didx, rows1, rows2, *sems):
    g1 = sems[:NV]
    g2 = sems[NV:2 * NV]
    ws = sems[2 * NV:]
    core = lax.axis_index("c")
    sub = lax.axis_index("s")
    tid = core * NS + sub
    row0 = tid * RT
    pltpu.sync_copy(src2.at[pl.ds(row0, RT)], sidx)
    pltpu.sync_copy(dst2.at[pl.ds(row0, RT)], didx)

    def g_start(c, b):
        pltpu.async_copy(asrc.at[sidx.at[c]], rows1.at[b], g1[b])
        pltpu.async_copy(adst.at[didx.at[c]], rows2.at[b], g2[b])

    def g_wait(b):
        pltpu.make_async_copy(asrc.at[pl.ds(0, CHUNK)], rows1.at[b],
                              g1[b]).wait()
        pltpu.make_async_copy(adst.at[pl.ds(0, CHUNK)], rows2.at[b],
                              g2[b]).wait()

    def w_start(c, b):
        pltpu.async_copy(rows1.at[b],
                         out_hbm.at[pl.ds((row0 + c) * CHUNK, CHUNK)], ws[b])

    def w_wait(b):
        pltpu.make_async_copy(rows1.at[b], out_hbm.at[pl.ds(0, CHUNK)],
                              ws[b]).wait()

    for b in range(NV):
        g_start(b, b)

    @pl.loop(0, RT // NV)
    def _(i):
        for b in range(NV):
            c = i * NV + b
            g_wait(b)
            _v_add(rows1.at[b], rows2.at[b])
            w_start(c, b)

            @pl.when(c + NV < RT)
            def _():
                w_wait(b)
                g_start(c + NV, b)

    for b in range(NV):
        w_wait(b)


def _vkern(a_src, a_dst, src2, dst2):
    return pl.kernel(
        _v_body,
        out_type=jax.ShapeDtypeStruct((EP, D), jnp.float32),
        mesh=_sc_mesh(),
        scratch_types=[
            pltpu.VMEM((RT, CHUNK), jnp.int32),
            pltpu.VMEM((RT, CHUNK), jnp.int32),
            pltpu.VMEM((NV, CHUNK, D), jnp.float32),
            pltpu.VMEM((NV, CHUNK, D), jnp.float32),
        ] + [pltpu.SemaphoreType.DMA] * (3 * NV),
    )(a_src, a_dst, src2, dst2)


BM = 512
BE = 4000


def _enc_body(x_ref, w_ref, b_ref, o_ref):
    o_ref[...] = jnp.dot(x_ref[...], w_ref[...],
                         preferred_element_type=jnp.float32) + b_ref[...]


def _encode(x_pad, W, b_row):
    return pl.pallas_call(
        _enc_body,
        grid=(NPAD // BM,),
        in_specs=[
            pl.BlockSpec((BM, D), lambda i: (i, 0)),
            pl.BlockSpec((D, D), lambda i: (0, 0)),
            pl.BlockSpec((1, D), lambda i: (0, 0)),
        ],
        out_specs=pl.BlockSpec((BM, D), lambda i: (i, 0)),
        out_shape=jax.ShapeDtypeStruct((NPAD, D), jnp.float32),
    )(x_pad, W, b_row)


def _recip_body(d_ref, o_ref):
    deg = d_ref[0, :, 0:1] + d_ref[1, :, 0:1]
    o_ref[...] = 1.0 / jnp.maximum(deg, 1.0)


def _recip(deg2):
    return pl.pallas_call(
        _recip_body,
        grid=(NPAD // BM,),
        in_specs=[pl.BlockSpec((NC, BM, D), lambda i: (0, i, 0))],
        out_specs=pl.BlockSpec((BM, 1), lambda i: (i, 0)),
        out_shape=jax.ShapeDtypeStruct((NPAD, 1), jnp.float32),
    )(deg2)


def _layer_body(a_ref, r_ref, w_ref, b_ref, o_ref):
    z = (a_ref[0] + a_ref[1]) * r_ref[...]
    o_ref[...] = jax.nn.relu(
        jnp.dot(z, w_ref[...], preferred_element_type=jnp.float32)
        + b_ref[...])


def _layer(agg2, recip, W, b_row):
    return pl.pallas_call(
        _layer_body,
        grid=(NPAD // BM,),
        in_specs=[
            pl.BlockSpec((NC, BM, D), lambda i: (0, i, 0)),
            pl.BlockSpec((BM, 1), lambda i: (i, 0)),
            pl.BlockSpec((D, D), lambda i: (0, 0)),
            pl.BlockSpec((1, D), lambda i: (0, 0)),
        ],
        out_specs=pl.BlockSpec((BM, D), lambda i: (i, 0)),
        out_shape=jax.ShapeDtypeStruct((NPAD, D), jnp.float32),
    )(agg2, recip, W, b_row)


def _layer3_body(a_ref, r_ref, w_ref, b_ref, ws_ref, wd_ref, os_ref, od_ref):
    z = (a_ref[0] + a_ref[1]) * r_ref[...]
    h = jax.nn.relu(
        jnp.dot(z, w_ref[...], preferred_element_type=jnp.float32)
        + b_ref[...])
    os_ref[...] = jnp.dot(h, ws_ref[...], preferred_element_type=jnp.float32)
    od_ref[...] = jnp.dot(h, wd_ref[...], preferred_element_type=jnp.float32)


def _layer3(agg2, recip, W, b_row, W_src, W_dst):
    return pl.pallas_call(
        _layer3_body,
        grid=(NPAD // BM,),
        in_specs=[
            pl.BlockSpec((NC, BM, D), lambda i: (0, i, 0)),
            pl.BlockSpec((BM, 1), lambda i: (i, 0)),
            pl.BlockSpec((D, D), lambda i: (0, 0)),
            pl.BlockSpec((1, D), lambda i: (0, 0)),
            pl.BlockSpec((D, D), lambda i: (0, 0)),
            pl.BlockSpec((D, D), lambda i: (0, 0)),
        ],
        out_specs=[
            pl.BlockSpec((BM, D), lambda i: (i, 0)),
            pl.BlockSpec((BM, D), lambda i: (i, 0)),
        ],
        out_shape=[
            jax.ShapeDtypeStruct((NPAD, D), jnp.float32),
            jax.ShapeDtypeStruct((NPAD, D), jnp.float32),
        ],
    )(agg2, recip, W, b_row, W_src, W_dst)


def _score_body(v_ref, e_ref, we_ref, be_ref, wo_ref, bo_ref, o_ref):
    eh = jnp.dot(e_ref[...], we_ref[...],
                 preferred_element_type=jnp.float32) + be_ref[...]
    t = jax.nn.relu(v_ref[...] + eh)
    o_ref[...] = jnp.sum(t * wo_ref[...], axis=1, keepdims=True) + bo_ref[...]


def _score(V, e_full, W_edge, be_row, wo_row, bo_11):
    return pl.pallas_call(
        _score_body,
        grid=(E // BE,),
        in_specs=[
            pl.BlockSpec((BE, D), lambda i: (i, 0)),
            pl.BlockSpec((BE, DE), lambda i: (i, 0)),
            pl.BlockSpec((DE, D), lambda i: (0, 0)),
            pl.BlockSpec((1, D), lambda i: (0, 0)),
            pl.BlockSpec((1, D), lambda i: (0, 0)),
            pl.BlockSpec((1, 1), lambda i: (0, 0)),
        ],
        out_specs=pl.BlockSpec((BE, 1), lambda i: (i, 0)),
        out_shape=jax.ShapeDtypeStruct((E, 1), jnp.float32),
    )(V, e_full, W_edge, be_row, wo_row, bo_11)


def kernel(x, edge_index, e, W_node, b_node, W_edge, b_edge,
           W_gcn0, b_gcn0, W_gcn1, b_gcn1, W_gcn2, b_gcn2,
           W_src, W_dst, w_out, b_out):
    pad = N + jnp.arange(EP - E, dtype=jnp.int32) % (NPAD - N)
    src1 = jnp.concatenate([edge_index[0].astype(jnp.int32), pad])
    dst1 = jnp.concatenate([edge_index[1].astype(jnp.int32), pad])
    src2 = src1.reshape(ER, CHUNK)
    dst2 = dst1.reshape(ER, CHUNK)
    x_pad = jnp.pad(x, ((0, NPAD - N), (0, 0)))
    zeros128 = jnp.zeros((NPAD, D), jnp.float32)
    ones128 = jnp.ones((CHUNK, D), jnp.float32)

    deg2 = _deg(dst2, ones128, zeros128)
    h = _encode(x_pad, W_node, b_node.reshape(1, D))
    recip = _recip(deg2)
    for W, b in ((W_gcn0, b_gcn0), (W_gcn1, b_gcn1)):
        agg2 = _scatter(h, src2, dst1, zeros128)
        h = _layer(agg2, recip, W, b.reshape(1, D))
    agg2 = _scatter(h, src2, dst1, zeros128)
    a_src, a_dst = _layer3(agg2, recip, W_gcn2, b_gcn2.reshape(1, D),
                           W_src, W_dst)

    V = _vkern(a_src, a_dst, src2, dst2)
    scores = _score(V, e, W_edge, b_edge.reshape(1, D),
                    w_out.reshape(1, D), b_out.reshape(1, 1))
    return scores[:, 0]
